# baseline (speedup 1.0000x reference)
import functools

import jax
import jax.numpy as jnp
from jax import lax
from jax.experimental import pallas as pl
from jax.experimental.pallas import tpu as pltpu
from jax.experimental.pallas import tpu_sc as plsc

_VOCAB = 1000
_N = 1024 * 50
_NW = 32
_PER_W = _N // _NW
_CHUNK = 40
_NCHUNK = _PER_W // _CHUNK


def _make_gather():
    mesh = plsc.VectorSubcoreMesh(core_axis_name="c", subcore_axis_name="s")

    @functools.partial(
        pl.kernel,
        mesh=mesh,
        compiler_params=pltpu.CompilerParams(use_tc_tiling_on_sc=False),
        out_type=jax.ShapeDtypeStruct((_N, _VOCAB), jnp.float32),
        scratch_types=[
            pltpu.VMEM((_PER_W,), jnp.int32),
            pltpu.VMEM((_CHUNK, _VOCAB), jnp.float32),
            pltpu.SemaphoreType.DMA,
        ],
    )
    def gather_kernel(idx_hbm, w_hbm, out_hbm, idx_v, rows_v, sem):
        wid = lax.axis_index("s") * 2 + lax.axis_index("c")
        base = wid * _PER_W
        pltpu.sync_copy(idx_hbm.at[pl.ds(base, _PER_W)], idx_v)

        def body(c, _):
            off = c * _CHUNK
            pltpu.async_copy(
                w_hbm.at[idx_v.at[pl.ds(off, _CHUNK)]], rows_v, sem
            ).wait()
            pltpu.sync_copy(rows_v, out_hbm.at[pl.ds(base + off, _CHUNK)])
            return 0

        lax.fori_loop(0, _NCHUNK, body, 0)

    return gather_kernel


_gather = _make_gather()


def kernel(idx, W):
    idx_flat = idx.reshape(_N).astype(jnp.int32)
    out = _gather(idx_flat, W)
    return out.reshape(idx.shape[0], idx.shape[1], _VOCAB)


# speedup vs baseline: 1.0735x; 1.0735x over previous
import functools

import jax
import jax.numpy as jnp
from jax import lax
from jax.experimental import pallas as pl
from jax.experimental.pallas import tpu as pltpu
from jax.experimental.pallas import tpu_sc as plsc

_VOCAB = 1000
_N = 1024 * 50
_NW = 32
_PER_W = _N // _NW
_CHUNK = 32
_NCHUNK = _PER_W // _CHUNK
_NPAIR = _NCHUNK // 2


def _make_gather():
    mesh = plsc.VectorSubcoreMesh(core_axis_name="c", subcore_axis_name="s")

    @functools.partial(
        pl.kernel,
        mesh=mesh,
        compiler_params=pltpu.CompilerParams(use_tc_tiling_on_sc=False),
        out_type=jax.ShapeDtypeStruct((_N, _VOCAB), jnp.float32),
        scratch_types=[
            pltpu.VMEM_SHARED((_VOCAB, _VOCAB), jnp.float32),
            pltpu.VMEM((_PER_W,), jnp.int32),
            pltpu.VMEM((_CHUNK, _VOCAB), jnp.float32),
            pltpu.VMEM((_CHUNK, _VOCAB), jnp.float32),
            pltpu.SemaphoreType.DMA,
            pltpu.SemaphoreType.DMA,
            pltpu.SemaphoreType.DMA,
            pltpu.SemaphoreType.DMA,
        ],
    )
    def gather_kernel(idx_hbm, w_hbm, out_hbm, w_sh, idx_v, buf0, buf1,
                      sg0, sg1, ss0, ss1):
        sid = lax.axis_index("s")
        wid = sid * 2 + lax.axis_index("c")
        base = wid * _PER_W

        @pl.when(sid == 0)
        def _():
            pltpu.sync_copy(w_hbm, w_sh)

        pltpu.sync_copy(idx_hbm.at[pl.ds(base, _PER_W)], idx_v)
        plsc.subcore_barrier()

        def body(j, _):
            c0 = j * 2
            o0 = c0 * _CHUNK
            o1 = o0 + _CHUNK
            h0 = pltpu.async_copy(
                w_sh.at[idx_v.at[pl.ds(o0, _CHUNK)]], buf0, sg0)
            h1 = pltpu.async_copy(
                w_sh.at[idx_v.at[pl.ds(o1, _CHUNK)]], buf1, sg1)
            h0.wait()
            s0 = pltpu.async_copy(
                buf0, out_hbm.at[pl.ds(base + o0, _CHUNK)], ss0)
            h1.wait()
            s1 = pltpu.async_copy(
                buf1, out_hbm.at[pl.ds(base + o1, _CHUNK)], ss1)
            s0.wait()
            s1.wait()
            return 0

        lax.fori_loop(0, _NPAIR, body, 0)

    return gather_kernel


_gather = _make_gather()


def kernel(idx, W):
    idx_flat = idx.reshape(_N).astype(jnp.int32)
    out = _gather(idx_flat, W)
    return out.reshape(idx.shape[0], idx.shape[1], _VOCAB)
